# Initial kernel scaffold; baseline (speedup 1.0000x reference)
import jax
import jax.numpy as jnp
from jax.experimental import pallas as pl


def kernel(enc_x, W_enc, embed, W_out):
    raise NotImplementedError("write your pallas kernel here")


# speedup vs baseline: 1.1851x; 1.1851x over previous
import jax
import jax.numpy as jnp
from jax.experimental import pallas as pl
from jax.experimental.pallas import tpu as pltpu

_BS = 16
_D_MODEL = 512
_VOCAB = 100000
_STEPS = 20
_SOS = 1
_EOS = 2

_VBLK = 6272
_NBLK = 16


def _ctx_body(enc_ref, w_ref, out_ref):
    bs, enc_len, d_in = enc_ref.shape
    x = jnp.dot(enc_ref[...].reshape(bs * enc_len, d_in), w_ref[...],
                preferred_element_type=jnp.float32)
    out_ref[...] = jnp.mean(x.reshape(bs, enc_len, _D_MODEL), axis=1)


def _ctx_call(enc_x, W_enc):
    return pl.pallas_call(
        _ctx_body,
        out_shape=jax.ShapeDtypeStruct((_BS, _D_MODEL), jnp.float32),
    )(enc_x, W_enc)


def _step_body(h_ref, w_ref, arg_ref, lp_ref, m_scr, s_scr, a_scr):
    i = pl.program_id(0)

    @pl.when(i == 0)
    def _init():
        m_scr[...] = jnp.full_like(m_scr, -jnp.inf)
        s_scr[...] = jnp.zeros_like(s_scr)
        a_scr[...] = jnp.zeros_like(a_scr)

    logits = jnp.dot(h_ref[...], w_ref[...], preferred_element_type=jnp.float32)
    col = jax.lax.broadcasted_iota(jnp.int32, (_BS, _VBLK), 1) + i * _VBLK
    valid = col < _VOCAB
    logits = jnp.where(valid, logits, -jnp.inf)

    bm = jnp.max(logits, axis=1)
    cand = jnp.where(logits == bm[:, None], col, jnp.int32(2**30))
    barg = jnp.min(cand, axis=1)

    m_old = m_scr[0, :]
    m_new = jnp.maximum(m_old, bm)
    s_new = (s_scr[0, :] * jnp.exp(m_old - m_new)
             + jnp.sum(jnp.exp(logits - m_new[:, None]), axis=1))
    a_new = jnp.where(bm > m_old, barg, a_scr[0, :])

    m_scr[0, :] = m_new
    s_scr[0, :] = s_new
    a_scr[0, :] = a_new

    @pl.when(i == _NBLK - 1)
    def _emit():
        arg_ref[0, :] = a_new
        lp_ref[0, :] = -jnp.log(s_new)


def _step_call(h, W_out):
    sampled, step_lp = pl.pallas_call(
        _step_body,
        grid=(_NBLK,),
        in_specs=[
            pl.BlockSpec((_BS, _D_MODEL), lambda i: (0, 0)),
            pl.BlockSpec((_D_MODEL, _VBLK), lambda i: (0, i)),
        ],
        out_specs=[
            pl.BlockSpec((1, _BS), lambda i: (0, 0)),
            pl.BlockSpec((1, _BS), lambda i: (0, 0)),
        ],
        out_shape=[
            jax.ShapeDtypeStruct((1, _BS), jnp.int32),
            jax.ShapeDtypeStruct((1, _BS), jnp.float32),
        ],
        scratch_shapes=[
            pltpu.VMEM((1, _BS), jnp.float32),
            pltpu.VMEM((1, _BS), jnp.float32),
            pltpu.VMEM((1, _BS), jnp.int32),
        ],
    )(h, W_out)
    return sampled[0], step_lp[0]


def kernel(enc_x, W_enc, embed, W_out):
    ctx = _ctx_call(enc_x, W_enc)

    prev = jnp.full((_BS,), _SOS, dtype=jnp.int32)
    tokens = [prev]
    lps = [jnp.zeros((_BS,), dtype=jnp.float32)]
    for _t in range(_STEPS):
        h = jnp.take(embed, prev, axis=0) + ctx
        sampled, step_lp = _step_call(h, W_out)
        sampled = sampled.astype(jnp.int32)
        tokens.append(sampled)
        lps.append(step_lp)
        prev = sampled

    toks = jnp.stack(tokens, axis=1)
    lp = jnp.stack(lps, axis=1)

    t_idx = jnp.arange(1, _STEPS + 1, dtype=jnp.int32)[None, :]
    where_is_eos = jnp.min(
        jnp.where(toks[:, 1:] == _EOS, t_idx, _STEPS), axis=1).astype(jnp.int32)
    ar = jnp.arange(_STEPS + 1)[None, :]
    lp = jnp.where(ar > where_is_eos[:, None], 0.0, lp)
    return toks, lp.reshape(_BS, 1, _STEPS + 1)


# speedup vs baseline: 1.2774x; 1.0778x over previous
import jax
import jax.numpy as jnp
from jax.experimental import pallas as pl
from jax.experimental.pallas import tpu as pltpu

_BS = 16
_D_MODEL = 512
_VOCAB = 100000
_STEPS = 20
_SOS = 1
_EOS = 2

_VBLK = 6272
_NBLK = 16


def _ctx_body(enc_ref, w_ref, out_ref):
    bs, enc_len, d_in = enc_ref.shape
    x = jnp.dot(enc_ref[...].reshape(bs * enc_len, d_in), w_ref[...],
                preferred_element_type=jnp.float32)
    out_ref[...] = jnp.mean(x.reshape(bs, enc_len, _D_MODEL), axis=1)


def _ctx_call(enc_x, W_enc):
    return pl.pallas_call(
        _ctx_body,
        out_shape=jax.ShapeDtypeStruct((_BS, _D_MODEL), jnp.float32),
    )(enc_x, W_enc)


def _step_body(h_ref, w_ref, arg_ref, lp_ref, m_scr, s_scr, a_scr):
    i = pl.program_id(0)

    @pl.when(i == 0)
    def _init():
        m_scr[...] = jnp.full_like(m_scr, -jnp.inf)
        s_scr[...] = jnp.zeros_like(s_scr)
        a_scr[...] = jnp.zeros_like(a_scr)

    logits = jnp.dot(h_ref[...], w_ref[...], preferred_element_type=jnp.float32)
    col = jax.lax.broadcasted_iota(jnp.int32, (_BS, _VBLK), 1) + i * _VBLK
    valid = col < _VOCAB
    logits = jnp.where(valid, logits, -jnp.inf)

    bm = jnp.max(logits, axis=1)
    cand = jnp.where(logits == bm[:, None], col, jnp.int32(2**30))
    barg = jnp.min(cand, axis=1)

    m_old = m_scr[0, :]
    m_new = jnp.maximum(m_old, bm)
    s_new = (s_scr[0, :] * jnp.exp(m_old - m_new)
             + jnp.sum(jnp.exp(logits - m_new[:, None]), axis=1))
    a_new = jnp.where(bm > m_old, barg, a_scr[0, :])

    m_scr[0, :] = m_new
    s_scr[0, :] = s_new
    a_scr[0, :] = a_new

    @pl.when(i == _NBLK - 1)
    def _emit():
        arg_ref[0, :] = a_new
        lp_ref[0, :] = -jnp.log(s_new)


def _step_call(h, W_out):
    sampled, step_lp = pl.pallas_call(
        _step_body,
        grid=(_NBLK,),
        in_specs=[
            pl.BlockSpec((_BS, _D_MODEL), lambda i: (0, 0)),
            pl.BlockSpec((_D_MODEL, _VBLK), lambda i: (0, i)),
        ],
        out_specs=[
            pl.BlockSpec((1, _BS), lambda i: (0, 0)),
            pl.BlockSpec((1, _BS), lambda i: (0, 0)),
        ],
        out_shape=[
            jax.ShapeDtypeStruct((1, _BS), jnp.int32),
            jax.ShapeDtypeStruct((1, _BS), jnp.float32),
        ],
        scratch_shapes=[
            pltpu.VMEM((1, _BS), jnp.float32),
            pltpu.VMEM((1, _BS), jnp.float32),
            pltpu.VMEM((1, _BS), jnp.int32),
        ],
    )(h, W_out)
    return sampled[0], step_lp[0]


def kernel(enc_x, W_enc, embed, W_out):
    ctx = _ctx_call(enc_x, W_enc)

    def _compute(prev):
        h = jnp.take(embed, prev, axis=0) + ctx
        sampled, step_lp = _step_call(h, W_out)
        return sampled.astype(jnp.int32), step_lp

    prev = jnp.full((_BS,), _SOS, dtype=jnp.int32)
    tokens = [prev]
    lps = [jnp.zeros((_BS,), dtype=jnp.float32)]
    for t in range(_STEPS):
        prev = tokens[t]
        if t == 0:
            sampled, step_lp = _compute(prev)
        else:
            keys = jnp.stack(tokens[0:t], axis=1)
            val_s = jnp.stack(tokens[1:t + 1], axis=1)
            val_l = jnp.stack(lps[1:t + 1], axis=1)
            eq = keys == prev[:, None]
            found = jnp.any(eq, axis=1)
            idx = jnp.argmax(eq, axis=1)
            memo_s = jnp.take_along_axis(val_s, idx[:, None], axis=1)[:, 0]
            memo_l = jnp.take_along_axis(val_l, idx[:, None], axis=1)[:, 0]
            sampled, step_lp = jax.lax.cond(
                jnp.all(found),
                lambda: (memo_s, memo_l),
                lambda: _compute(prev))
        tokens.append(sampled)
        lps.append(step_lp)

    toks = jnp.stack(tokens, axis=1)
    lp = jnp.stack(lps, axis=1)

    t_idx = jnp.arange(1, _STEPS + 1, dtype=jnp.int32)[None, :]
    where_is_eos = jnp.min(
        jnp.where(toks[:, 1:] == _EOS, t_idx, _STEPS), axis=1).astype(jnp.int32)
    ar = jnp.arange(_STEPS + 1)[None, :]
    lp = jnp.where(ar > where_is_eos[:, None], 0.0, lp)
    return toks, lp.reshape(_BS, 1, _STEPS + 1)


# speedup vs baseline: 3.4136x; 2.6723x over previous
import jax
import jax.numpy as jnp
from jax.experimental import pallas as pl
from jax.experimental.pallas import tpu as pltpu

_BS = 16
_D_MODEL = 512
_VOCAB = 100000
_STEPS = 20
_SOS = 1
_EOS = 2

_VBLK = 6272
_NBLK = 16
_LAST = _VOCAB - (_NBLK - 1) * _VBLK
_HCOLS = 32
_NEG_INF = float("-inf")


def _ctx_body(enc_ref, w_ref, out_ref):
    bs, enc_len, d_in = enc_ref.shape
    x = jnp.dot(enc_ref[...].reshape(bs * enc_len, d_in), w_ref[...],
                preferred_element_type=jnp.float32)
    out_ref[...] = jnp.mean(x.reshape(bs, enc_len, _D_MODEL), axis=1)


def _ctx_call(enc_x, W_enc):
    return pl.pallas_call(
        _ctx_body,
        out_shape=jax.ShapeDtypeStruct((_BS, _D_MODEL), jnp.float32),
    )(enc_x, W_enc)


def _decode_body(ctx_ref, emb_hbm, w_hbm, tok_ref, lp_ref,
                 wb0, wb1, wlast, hbuf, vec_v, vec_s,
                 keyh, valh, lph, res_s, res_l,
                 sem_w, sem_l, sem_h, sem_p):
    lanes = jax.lax.broadcasted_iota(jnp.int32, (_BS, _HCOLS), 1)

    tok_ref[...] = jnp.where(lanes == 0, jnp.int32(_SOS), 0)
    lp_ref[...] = jnp.zeros_like(lp_ref)
    keyh[...] = jnp.full_like(keyh, -1)
    valh[...] = jnp.zeros_like(valh)
    lph[...] = jnp.zeros_like(lph)

    def _w_copy(i):
        if i < _NBLK - 1:
            return pltpu.make_async_copy(
                w_hbm.at[:, pl.ds(i * _VBLK, _VBLK)],
                wb0 if i % 2 == 0 else wb1,
                sem_w.at[i % 2])
        return pltpu.make_async_copy(
            w_hbm.at[:, pl.ds(i * _VBLK, _LAST)], wlast, sem_l)

    def _step(t, prev):
        eq = (keyh[...] == prev[:, None]) & (lanes < t)
        found = jnp.any(eq, axis=1)
        res_s[0, :] = jnp.min(jnp.where(eq, valh[...], jnp.int32(2**30)), axis=1)
        res_l[0, :] = jnp.min(jnp.where(eq, lph[...], jnp.inf), axis=1)

        vec_v[0, :] = prev
        vec_v[1, :] = found.astype(jnp.int32)
        stage = pltpu.make_async_copy(vec_v, vec_s, sem_p)
        stage.start()
        stage.wait()
        n_found = vec_s[1, 0]
        for b in range(1, _BS):
            n_found = n_found + vec_s[1, b]
        need = n_found < _BS

        @pl.when(need)
        def _sweep():
            copies = [pltpu.make_async_copy(
                emb_hbm.at[pl.ds(vec_s[0, b], 1), :],
                hbuf.at[pl.ds(b, 1), :], sem_h) for b in range(_BS)]
            for c in copies:
                c.start()
            for c in copies:
                c.wait()
            h = hbuf[...] + ctx_ref[...]

            _w_copy(0).start()
            _w_copy(1).start()
            m = jnp.full((_BS,), _NEG_INF, dtype=jnp.float32)
            s = jnp.zeros((_BS,), dtype=jnp.float32)
            a = jnp.zeros((_BS,), dtype=jnp.int32)
            for i in range(_NBLK):
                _w_copy(i).wait()
                if i == _NBLK - 1:
                    w = wlast[...]
                else:
                    w = (wb0 if i % 2 == 0 else wb1)[...]
                nblk_cols = w.shape[1]
                logits = jnp.dot(h, w, preferred_element_type=jnp.float32)
                col = jax.lax.broadcasted_iota(
                    jnp.int32, (_BS, nblk_cols), 1) + i * _VBLK
                bm = jnp.max(logits, axis=1)
                cand = jnp.where(logits == bm[:, None], col, jnp.int32(2**30))
                barg = jnp.min(cand, axis=1)
                m_new = jnp.maximum(m, bm)
                s = (s * jnp.exp(m - m_new)
                     + jnp.sum(jnp.exp(logits - m_new[:, None]), axis=1))
                a = jnp.where(bm > m, barg, a)
                m = m_new
                if i + 2 < _NBLK:
                    _w_copy(i + 2).start()
            res_s[0, :] = a
            res_l[0, :] = -jnp.log(s)

        sampled = res_s[0, :]
        step_lp = res_l[0, :]
        keyh[...] = jnp.where(lanes == t, prev[:, None], keyh[...])
        valh[...] = jnp.where(lanes == t, sampled[:, None], valh[...])
        lph[...] = jnp.where(lanes == t, step_lp[:, None], lph[...])
        tok_ref[...] = jnp.where(lanes == t + 1, sampled[:, None], tok_ref[...])
        lp_ref[...] = jnp.where(lanes == t + 1, step_lp[:, None], lp_ref[...])
        return sampled

    prev0 = jnp.full((_BS,), _SOS, dtype=jnp.int32)
    jax.lax.fori_loop(0, _STEPS, _step, prev0)


def kernel(enc_x, W_enc, embed, W_out):
    ctx = _ctx_call(enc_x, W_enc)

    toks32, lp32 = pl.pallas_call(
        _decode_body,
        in_specs=[
            pl.BlockSpec(memory_space=pltpu.VMEM),
            pl.BlockSpec(memory_space=pl.ANY),
            pl.BlockSpec(memory_space=pl.ANY),
        ],
        out_specs=[
            pl.BlockSpec(memory_space=pltpu.VMEM),
            pl.BlockSpec(memory_space=pltpu.VMEM),
        ],
        out_shape=[
            jax.ShapeDtypeStruct((_BS, _HCOLS), jnp.int32),
            jax.ShapeDtypeStruct((_BS, _HCOLS), jnp.float32),
        ],
        scratch_shapes=[
            pltpu.VMEM((_D_MODEL, _VBLK), jnp.float32),
            pltpu.VMEM((_D_MODEL, _VBLK), jnp.float32),
            pltpu.VMEM((_D_MODEL, _LAST), jnp.float32),
            pltpu.VMEM((_BS, _D_MODEL), jnp.float32),
            pltpu.VMEM((2, _BS), jnp.int32),
            pltpu.SMEM((2, _BS), jnp.int32),
            pltpu.VMEM((_BS, _HCOLS), jnp.int32),
            pltpu.VMEM((_BS, _HCOLS), jnp.int32),
            pltpu.VMEM((_BS, _HCOLS), jnp.float32),
            pltpu.VMEM((1, _BS), jnp.int32),
            pltpu.VMEM((1, _BS), jnp.float32),
            pltpu.SemaphoreType.DMA((2,)),
            pltpu.SemaphoreType.DMA,
            pltpu.SemaphoreType.DMA,
            pltpu.SemaphoreType.DMA,
        ],
    )(ctx, embed, W_out)

    toks = toks32[:, :_STEPS + 1]
    lp = lp32[:, :_STEPS + 1]
    t_idx = jnp.arange(1, _STEPS + 1, dtype=jnp.int32)[None, :]
    where_is_eos = jnp.min(
        jnp.where(toks[:, 1:] == _EOS, t_idx, _STEPS), axis=1).astype(jnp.int32)
    ar = jnp.arange(_STEPS + 1)[None, :]
    lp = jnp.where(ar > where_is_eos[:, None], 0.0, lp)
    return toks, lp.reshape(_BS, 1, _STEPS + 1)


# speedup vs baseline: 3.4421x; 1.0084x over previous
import jax
import jax.numpy as jnp
from jax.experimental import pallas as pl
from jax.experimental.pallas import tpu as pltpu

_BS = 16
_ENC_LEN = 49
_D_IN = 1024
_D_MODEL = 512
_VOCAB = 100000
_STEPS = 20
_SOS = 1
_EOS = 2

_VBLK = 6272
_NBLK = 16
_LAST = _VOCAB - (_NBLK - 1) * _VBLK
_HCOLS = 32
_NEG_INF = float("-inf")


def _decode_body(enc_ref, wenc_ref, emb_hbm, w_hbm, tok_ref, lp_ref,
                 ctx_s, wb0, wb1, wlast, hbuf, vec_v, vec_s,
                 keyh, valh, lph, res_s, res_l, tokbuf, lpbuf,
                 sem_w, sem_l, sem_h, sem_p):
    lanes = jax.lax.broadcasted_iota(jnp.int32, (_BS, _HCOLS), 1)

    x = jnp.dot(enc_ref[...].reshape(_BS * _ENC_LEN, _D_IN), wenc_ref[...],
                preferred_element_type=jnp.float32)
    ctx_s[...] = jnp.mean(x.reshape(_BS, _ENC_LEN, _D_MODEL), axis=1)

    tokbuf[...] = jnp.where(lanes == 0, jnp.int32(_SOS), 0)
    lpbuf[...] = jnp.zeros_like(lpbuf)
    keyh[...] = jnp.full_like(keyh, -1)
    valh[...] = jnp.zeros_like(valh)
    lph[...] = jnp.zeros_like(lph)

    def _w_copy(i):
        if i < _NBLK - 1:
            return pltpu.make_async_copy(
                w_hbm.at[:, pl.ds(i * _VBLK, _VBLK)],
                wb0 if i % 2 == 0 else wb1,
                sem_w.at[i % 2])
        return pltpu.make_async_copy(
            w_hbm.at[:, pl.ds(i * _VBLK, _LAST)], wlast, sem_l)

    def _step(t, carry):
        prev, eosv = carry
        eq = (keyh[...] == prev[:, None]) & (lanes < t)
        found = jnp.any(eq, axis=1)
        res_s[0, :] = jnp.min(jnp.where(eq, valh[...], jnp.int32(2**30)), axis=1)
        res_l[0, :] = jnp.min(jnp.where(eq, lph[...], jnp.inf), axis=1)

        vec_v[0, :] = prev
        vec_v[1, :] = found.astype(jnp.int32)
        stage = pltpu.make_async_copy(vec_v, vec_s, sem_p)
        stage.start()
        stage.wait()
        n_found = vec_s[1, 0]
        for b in range(1, _BS):
            n_found = n_found + vec_s[1, b]
        need = n_found < _BS

        @pl.when(need)
        def _sweep():
            copies = [pltpu.make_async_copy(
                emb_hbm.at[pl.ds(vec_s[0, b], 1), :],
                hbuf.at[pl.ds(b, 1), :], sem_h) for b in range(_BS)]
            for c in copies:
                c.start()
            for c in copies:
                c.wait()
            h = hbuf[...] + ctx_s[...]

            _w_copy(0).start()
            _w_copy(1).start()
            m = jnp.full((_BS,), _NEG_INF, dtype=jnp.float32)
            s = jnp.zeros((_BS,), dtype=jnp.float32)
            a = jnp.zeros((_BS,), dtype=jnp.int32)
            for i in range(_NBLK):
                _w_copy(i).wait()
                if i == _NBLK - 1:
                    w = wlast[...]
                else:
                    w = (wb0 if i % 2 == 0 else wb1)[...]
                logits = jnp.dot(h, w, preferred_element_type=jnp.float32)
                col = jax.lax.broadcasted_iota(
                    jnp.int32, (_BS, w.shape[1]), 1) + i * _VBLK
                bm = jnp.max(logits, axis=1)
                cand = jnp.where(logits == bm[:, None], col, jnp.int32(2**30))
                barg = jnp.min(cand, axis=1)
                m_new = jnp.maximum(m, bm)
                s = (s * jnp.exp(m - m_new)
                     + jnp.sum(jnp.exp(logits - m_new[:, None]), axis=1))
                a = jnp.where(bm > m, barg, a)
                m = m_new
                if i + 2 < _NBLK:
                    _w_copy(i + 2).start()
            res_s[0, :] = a
            res_l[0, :] = -jnp.log(s)

        sampled = res_s[0, :]
        step_lp = res_l[0, :]
        keyh[...] = jnp.where(lanes == t, prev[:, None], keyh[...])
        valh[...] = jnp.where(lanes == t, sampled[:, None], valh[...])
        lph[...] = jnp.where(lanes == t, step_lp[:, None], lph[...])
        tokbuf[...] = jnp.where(lanes == t + 1, sampled[:, None], tokbuf[...])
        lpbuf[...] = jnp.where(lanes == t + 1, step_lp[:, None], lpbuf[...])
        eosv = jnp.minimum(
            eosv, jnp.where(sampled == _EOS, t + 1, _STEPS).astype(jnp.int32))
        return sampled, eosv

    prev0 = jnp.full((_BS,), _SOS, dtype=jnp.int32)
    eos0 = jnp.full((_BS,), _STEPS, dtype=jnp.int32)
    _, eosv = jax.lax.fori_loop(0, _STEPS, _step, (prev0, eos0))

    tok_ref[...] = tokbuf[:, :_STEPS + 1]
    lp_masked = jnp.where(lanes > eosv[:, None], 0.0, lpbuf[...])
    lp_ref[...] = lp_masked[:, :_STEPS + 1]


def kernel(enc_x, W_enc, embed, W_out):
    toks, lp = pl.pallas_call(
        _decode_body,
        in_specs=[
            pl.BlockSpec(memory_space=pltpu.VMEM),
            pl.BlockSpec(memory_space=pltpu.VMEM),
            pl.BlockSpec(memory_space=pl.ANY),
            pl.BlockSpec(memory_space=pl.ANY),
        ],
        out_specs=[
            pl.BlockSpec(memory_space=pltpu.VMEM),
            pl.BlockSpec(memory_space=pltpu.VMEM),
        ],
        out_shape=[
            jax.ShapeDtypeStruct((_BS, _STEPS + 1), jnp.int32),
            jax.ShapeDtypeStruct((_BS, _STEPS + 1), jnp.float32),
        ],
        scratch_shapes=[
            pltpu.VMEM((_BS, _D_MODEL), jnp.float32),
            pltpu.VMEM((_D_MODEL, _VBLK), jnp.float32),
            pltpu.VMEM((_D_MODEL, _VBLK), jnp.float32),
            pltpu.VMEM((_D_MODEL, _LAST), jnp.float32),
            pltpu.VMEM((_BS, _D_MODEL), jnp.float32),
            pltpu.VMEM((2, _BS), jnp.int32),
            pltpu.SMEM((2, _BS), jnp.int32),
            pltpu.VMEM((_BS, _HCOLS), jnp.int32),
            pltpu.VMEM((_BS, _HCOLS), jnp.int32),
            pltpu.VMEM((_BS, _HCOLS), jnp.float32),
            pltpu.VMEM((1, _BS), jnp.int32),
            pltpu.VMEM((1, _BS), jnp.float32),
            pltpu.VMEM((_BS, _HCOLS), jnp.int32),
            pltpu.VMEM((_BS, _HCOLS), jnp.float32),
            pltpu.SemaphoreType.DMA((2,)),
            pltpu.SemaphoreType.DMA,
            pltpu.SemaphoreType.DMA,
            pltpu.SemaphoreType.DMA,
        ],
    )(enc_x, W_enc, embed, W_out)
    return toks, lp.reshape(_BS, 1, _STEPS + 1)


# speedup vs baseline: 3.4878x; 1.0133x over previous
import jax
import jax.numpy as jnp
from jax.experimental import pallas as pl
from jax.experimental.pallas import tpu as pltpu

_BS = 16
_ENC_LEN = 49
_D_IN = 1024
_D_MODEL = 512
_VOCAB = 100000
_STEPS = 20
_SOS = 1
_EOS = 2

_VBLK = 6272
_NBLK = 16
_LAST = _VOCAB - (_NBLK - 1) * _VBLK
_HCOLS = 32
_NEG_INF = float("-inf")


def _decode_body(enc_ref, wenc_ref, emb_hbm, w_hbm, tok_ref, lp_ref,
                 ctx_s, wb0, wb1, wlast, hbuf, vec_v, vec_s,
                 keyh, valh, lph, res_s, res_l, tokbuf, lpbuf,
                 sem_w, sem_l, sem_h, sem_p):
    lanes = jax.lax.broadcasted_iota(jnp.int32, (_BS, _HCOLS), 1)

    x = jnp.dot(enc_ref[...].reshape(_BS * _ENC_LEN, _D_IN), wenc_ref[...],
                preferred_element_type=jnp.float32)
    ctx_s[...] = jnp.mean(x.reshape(_BS, _ENC_LEN, _D_MODEL), axis=1)

    tokbuf[...] = jnp.where(lanes == 0, jnp.int32(_SOS), 0)
    lpbuf[...] = jnp.zeros_like(lpbuf)
    keyh[...] = jnp.full_like(keyh, -1)
    valh[...] = jnp.zeros_like(valh)
    lph[...] = jnp.zeros_like(lph)

    def _w_copy(i):
        if i < _NBLK - 1:
            return pltpu.make_async_copy(
                w_hbm.at[:, pl.ds(i * _VBLK, _VBLK)],
                wb0 if i % 2 == 0 else wb1,
                sem_w.at[i % 2])
        return pltpu.make_async_copy(
            w_hbm.at[:, pl.ds(i * _VBLK, _LAST)], wlast, sem_l)

    def _step(t, carry):
        prev, eosv, replay = carry
        eq = (keyh[...] == prev[:, None]) & (lanes < t)
        found = jnp.any(eq, axis=1)
        res_s[0, :] = jnp.min(jnp.where(eq, valh[...], jnp.int32(2**30)), axis=1)
        res_l[0, :] = jnp.min(jnp.where(eq, lph[...], jnp.inf), axis=1)

        @pl.when(replay == 0)
        def _stage():
            vec_v[0, :] = prev
            vec_v[1, :] = found.astype(jnp.int32)
            stage = pltpu.make_async_copy(vec_v, vec_s, sem_p)
            stage.start()
            stage.wait()

        n_found = vec_s[1, 0]
        for b in range(1, _BS):
            n_found = n_found + vec_s[1, b]
        need = n_found < _BS

        @pl.when(need)
        def _sweep():
            copies = [pltpu.make_async_copy(
                emb_hbm.at[pl.ds(vec_s[0, b], 1), :],
                hbuf.at[pl.ds(b, 1), :], sem_h) for b in range(_BS)]
            for c in copies:
                c.start()
            for c in copies:
                c.wait()
            h = hbuf[...] + ctx_s[...]

            _w_copy(0).start()
            _w_copy(1).start()
            m = jnp.full((_BS,), _NEG_INF, dtype=jnp.float32)
            s = jnp.zeros((_BS,), dtype=jnp.float32)
            a = jnp.zeros((_BS,), dtype=jnp.int32)
            for i in range(_NBLK):
                _w_copy(i).wait()
                if i == _NBLK - 1:
                    w = wlast[...]
                else:
                    w = (wb0 if i % 2 == 0 else wb1)[...]
                logits = jnp.dot(h, w, preferred_element_type=jnp.float32)
                col = jax.lax.broadcasted_iota(
                    jnp.int32, (_BS, w.shape[1]), 1) + i * _VBLK
                bm = jnp.max(logits, axis=1)
                cand = jnp.where(logits == bm[:, None], col, jnp.int32(2**30))
                barg = jnp.min(cand, axis=1)
                m_new = jnp.maximum(m, bm)
                s = (s * jnp.exp(m - m_new)
                     + jnp.sum(jnp.exp(logits - m_new[:, None]), axis=1))
                a = jnp.where(bm > m, barg, a)
                m = m_new
                if i + 2 < _NBLK:
                    _w_copy(i + 2).start()
            res_s[0, :] = a
            res_l[0, :] = -jnp.log(s)

        sampled = res_s[0, :]
        step_lp = res_l[0, :]
        keyh[...] = jnp.where(lanes == t, prev[:, None], keyh[...])
        valh[...] = jnp.where(lanes == t, sampled[:, None], valh[...])
        lph[...] = jnp.where(lanes == t, step_lp[:, None], lph[...])
        tokbuf[...] = jnp.where(lanes == t + 1, sampled[:, None], tokbuf[...])
        lpbuf[...] = jnp.where(lanes == t + 1, step_lp[:, None], lpbuf[...])
        eosv = jnp.minimum(
            eosv, jnp.where(sampled == _EOS, t + 1, _STEPS).astype(jnp.int32))
        replay = jnp.where(n_found == _BS, jnp.int32(1), replay)
        return sampled, eosv, replay

    prev0 = jnp.full((_BS,), _SOS, dtype=jnp.int32)
    eos0 = jnp.full((_BS,), _STEPS, dtype=jnp.int32)
    _, eosv, _ = jax.lax.fori_loop(
        0, _STEPS, _step, (prev0, eos0, jnp.int32(0)))

    tok_ref[...] = tokbuf[:, :_STEPS + 1]
    lp_masked = jnp.where(lanes > eosv[:, None], 0.0, lpbuf[...])
    lp_ref[...] = lp_masked[:, :_STEPS + 1]


def kernel(enc_x, W_enc, embed, W_out):
    toks, lp = pl.pallas_call(
        _decode_body,
        in_specs=[
            pl.BlockSpec(memory_space=pltpu.VMEM),
            pl.BlockSpec(memory_space=pltpu.VMEM),
            pl.BlockSpec(memory_space=pl.ANY),
            pl.BlockSpec(memory_space=pl.ANY),
        ],
        out_specs=[
            pl.BlockSpec(memory_space=pltpu.VMEM),
            pl.BlockSpec(memory_space=pltpu.VMEM),
        ],
        out_shape=[
            jax.ShapeDtypeStruct((_BS, _STEPS + 1), jnp.int32),
            jax.ShapeDtypeStruct((_BS, _STEPS + 1), jnp.float32),
        ],
        scratch_shapes=[
            pltpu.VMEM((_BS, _D_MODEL), jnp.float32),
            pltpu.VMEM((_D_MODEL, _VBLK), jnp.float32),
            pltpu.VMEM((_D_MODEL, _VBLK), jnp.float32),
            pltpu.VMEM((_D_MODEL, _LAST), jnp.float32),
            pltpu.VMEM((_BS, _D_MODEL), jnp.float32),
            pltpu.VMEM((2, _BS), jnp.int32),
            pltpu.SMEM((2, _BS), jnp.int32),
            pltpu.VMEM((_BS, _HCOLS), jnp.int32),
            pltpu.VMEM((_BS, _HCOLS), jnp.int32),
            pltpu.VMEM((_BS, _HCOLS), jnp.float32),
            pltpu.VMEM((1, _BS), jnp.int32),
            pltpu.VMEM((1, _BS), jnp.float32),
            pltpu.VMEM((_BS, _HCOLS), jnp.int32),
            pltpu.VMEM((_BS, _HCOLS), jnp.float32),
            pltpu.SemaphoreType.DMA((2,)),
            pltpu.SemaphoreType.DMA,
            pltpu.SemaphoreType.DMA,
            pltpu.SemaphoreType.DMA,
        ],
    )(enc_x, W_enc, embed, W_out)
    return toks, lp.reshape(_BS, 1, _STEPS + 1)


# speedup vs baseline: 5.3214x; 1.5257x over previous
import jax
import jax.numpy as jnp
from jax.experimental import pallas as pl
from jax.experimental.pallas import tpu as pltpu

_BS = 16
_ENC_LEN = 49
_D_IN = 1024
_D_MODEL = 512
_VOCAB = 100000
_STEPS = 20
_SOS = 1
_EOS = 2

_VBLK = 6272
_NBLK = 16
_LAST = _VOCAB - (_NBLK - 1) * _VBLK
_HCOLS = 32
_NEG_INF = float("-inf")


def _decode_body(enc_ref, wenc_ref, emb_hbm, w_hbm, tok_ref, lp_ref,
                 ctx_s, wb0, wb1, wlast, hbuf, vec_v, vec_s,
                 keyh, valh, lph, res_s, res_l, tokbuf, lpbuf,
                 sem_w, sem_l, sem_h, sem_p):
    lanes = jax.lax.broadcasted_iota(jnp.int32, (_BS, _HCOLS), 1)

    x = jnp.dot(enc_ref[...].reshape(_BS * _ENC_LEN, _D_IN), wenc_ref[...],
                preferred_element_type=jnp.float32)
    ctx_s[...] = jnp.mean(x.reshape(_BS, _ENC_LEN, _D_MODEL), axis=1)

    tokbuf[...] = jnp.where(lanes == 0, jnp.int32(_SOS), 0)
    lpbuf[...] = jnp.zeros_like(lpbuf)
    keyh[...] = jnp.full_like(keyh, -1)
    valh[...] = jnp.zeros_like(valh)
    lph[...] = jnp.zeros_like(lph)

    def _w_copy(i):
        if i < _NBLK - 1:
            return pltpu.make_async_copy(
                w_hbm.at[pl.ds(i * _VBLK, _VBLK), :],
                wb0 if i % 2 == 0 else wb1,
                sem_w.at[i % 2])
        return pltpu.make_async_copy(
            w_hbm.at[pl.ds(i * _VBLK, _LAST), :], wlast, sem_l)

    def _step(t, carry):
        prev, eosv, replay = carry
        eq = (keyh[...] == prev[:, None]) & (lanes < t)
        found = jnp.any(eq, axis=1)
        res_s[0, :] = jnp.min(jnp.where(eq, valh[...], jnp.int32(2**30)), axis=1)
        res_l[0, :] = jnp.min(jnp.where(eq, lph[...], jnp.inf), axis=1)

        @pl.when(replay == 0)
        def _stage():
            vec_v[0, :] = prev
            vec_v[1, :] = found.astype(jnp.int32)
            stage = pltpu.make_async_copy(vec_v, vec_s, sem_p)
            stage.start()
            stage.wait()

        n_found = vec_s[1, 0]
        for b in range(1, _BS):
            n_found = n_found + vec_s[1, b]
        need = n_found < _BS

        @pl.when(need)
        def _sweep():
            copies = [pltpu.make_async_copy(
                emb_hbm.at[pl.ds(vec_s[0, b], 1), :],
                hbuf.at[pl.ds(b, 1), :], sem_h) for b in range(_BS)]
            for c in copies:
                c.start()
            for c in copies:
                c.wait()
            h = hbuf[...] + ctx_s[...]

            _w_copy(0).start()
            _w_copy(1).start()
            m = jnp.full((_BS,), _NEG_INF, dtype=jnp.float32)
            s = jnp.zeros((_BS,), dtype=jnp.float32)
            a = jnp.zeros((_BS,), dtype=jnp.int32)
            for i in range(_NBLK):
                _w_copy(i).wait()
                if i == _NBLK - 1:
                    w = wlast[...]
                else:
                    w = (wb0 if i % 2 == 0 else wb1)[...]
                logits = jax.lax.dot_general(
                    h, w, (((1,), (1,)), ((), ())),
                    preferred_element_type=jnp.float32)
                col = jax.lax.broadcasted_iota(
                    jnp.int32, (_BS, w.shape[0]), 1) + i * _VBLK
                bm = jnp.max(logits, axis=1)
                cand = jnp.where(logits == bm[:, None], col, jnp.int32(2**30))
                barg = jnp.min(cand, axis=1)
                m_new = jnp.maximum(m, bm)
                s = (s * jnp.exp(m - m_new)
                     + jnp.sum(jnp.exp(logits - m_new[:, None]), axis=1))
                a = jnp.where(bm > m, barg, a)
                m = m_new
                if i + 2 < _NBLK:
                    _w_copy(i + 2).start()
            res_s[0, :] = a
            res_l[0, :] = -jnp.log(s)

        sampled = res_s[0, :]
        step_lp = res_l[0, :]
        keyh[...] = jnp.where(lanes == t, prev[:, None], keyh[...])
        valh[...] = jnp.where(lanes == t, sampled[:, None], valh[...])
        lph[...] = jnp.where(lanes == t, step_lp[:, None], lph[...])
        tokbuf[...] = jnp.where(lanes == t + 1, sampled[:, None], tokbuf[...])
        lpbuf[...] = jnp.where(lanes == t + 1, step_lp[:, None], lpbuf[...])
        eosv = jnp.minimum(
            eosv, jnp.where(sampled == _EOS, t + 1, _STEPS).astype(jnp.int32))
        replay = jnp.where(n_found == _BS, jnp.int32(1), replay)
        return sampled, eosv, replay

    prev0 = jnp.full((_BS,), _SOS, dtype=jnp.int32)
    eos0 = jnp.full((_BS,), _STEPS, dtype=jnp.int32)
    _, eosv, _ = jax.lax.fori_loop(
        0, _STEPS, _step, (prev0, eos0, jnp.int32(0)))

    tok_ref[...] = tokbuf[:, :_STEPS + 1]
    lp_masked = jnp.where(lanes > eosv[:, None], 0.0, lpbuf[...])
    lp_ref[...] = lp_masked[:, :_STEPS + 1]


def kernel(enc_x, W_enc, embed, W_out):
    toks, lp = pl.pallas_call(
        _decode_body,
        in_specs=[
            pl.BlockSpec(memory_space=pltpu.VMEM),
            pl.BlockSpec(memory_space=pltpu.VMEM),
            pl.BlockSpec(memory_space=pl.ANY),
            pl.BlockSpec(memory_space=pl.ANY),
        ],
        out_specs=[
            pl.BlockSpec(memory_space=pltpu.VMEM),
            pl.BlockSpec(memory_space=pltpu.VMEM),
        ],
        out_shape=[
            jax.ShapeDtypeStruct((_BS, _STEPS + 1), jnp.int32),
            jax.ShapeDtypeStruct((_BS, _STEPS + 1), jnp.float32),
        ],
        scratch_shapes=[
            pltpu.VMEM((_BS, _D_MODEL), jnp.float32),
            pltpu.VMEM((_VBLK, _D_MODEL), jnp.float32),
            pltpu.VMEM((_VBLK, _D_MODEL), jnp.float32),
            pltpu.VMEM((_LAST, _D_MODEL), jnp.float32),
            pltpu.VMEM((_BS, _D_MODEL), jnp.float32),
            pltpu.VMEM((2, _BS), jnp.int32),
            pltpu.SMEM((2, _BS), jnp.int32),
            pltpu.VMEM((_BS, _HCOLS), jnp.int32),
            pltpu.VMEM((_BS, _HCOLS), jnp.int32),
            pltpu.VMEM((_BS, _HCOLS), jnp.float32),
            pltpu.VMEM((1, _BS), jnp.int32),
            pltpu.VMEM((1, _BS), jnp.float32),
            pltpu.VMEM((_BS, _HCOLS), jnp.int32),
            pltpu.VMEM((_BS, _HCOLS), jnp.float32),
            pltpu.SemaphoreType.DMA((2,)),
            pltpu.SemaphoreType.DMA,
            pltpu.SemaphoreType.DMA,
            pltpu.SemaphoreType.DMA,
        ],
    )(enc_x, W_enc, embed, W_out.T)
    return toks, lp.reshape(_BS, 1, _STEPS + 1)
